# baseline (speedup 1.0000x reference)
import functools

import jax
import jax.numpy as jnp
from jax import lax
from jax.experimental import pallas as pl
from jax.experimental.pallas import tpu as pltpu
from jax.experimental.pallas import tpu_sc as plsc

NUM_PARTITIONS = 8
P = NUM_PARTITIONS
NW = 32
CHUNK = 16


def _sc_body(Tw, K, D, x_hbm, idx_hbm, out_hbm,
             xbuf, idxbuf, zbuf, xsem, wsem):
    nchunks = Tw // CHUNK
    wid = lax.axis_index("s") * 2 + lax.axis_index("c")
    tbase = wid * Tw

    for r in range(P - 1):
        for v in range(D // 16):
            zbuf[r, pl.ds(v * 16, 16)] = jnp.zeros((16,), jnp.float32)

    pltpu.sync_copy(idx_hbm.at[pl.ds(tbase * K, Tw * K)],
                    idxbuf.at[pl.ds(0, Tw * K)])

    def load_chunk(ci):
        pltpu.make_async_copy(
            x_hbm.at[pl.ds(tbase + ci * CHUNK, CHUNK)],
            xbuf.at[ci % 2], xsem).start()

    def wait_chunk(ci):
        pltpu.make_async_copy(
            x_hbm.at[pl.ds(tbase + ci * CHUNK, CHUNK)],
            xbuf.at[ci % 2], xsem).wait()

    def drain_rows(n):
        def body(j, carry):
            pltpu.make_async_copy(
                x_hbm.at[pl.ds(0, 1)], zbuf.at[pl.ds(0, 1)], wsem).wait()
            return carry
        lax.fori_loop(0, n, body, 0)

    def zero_run(base, start, length, max_len):
        for L in range(1, max_len + 1):
            @pl.when(length == L)
            def _(L=L):
                pltpu.make_async_copy(
                    zbuf.at[pl.ds(0, L)],
                    out_hbm.at[pl.ds(base + start, L)],
                    wsem).start()

    load_chunk(0)
    for ci in range(nchunks):
        wait_chunk(ci)
        slot = ci % 2

        def tok_body(t, carry, ci=ci, slot=slot):
            tloc = ci * CHUNK + t
            g = tbase + tloc
            iv = idxbuf[pl.ds(tloc * K, 16)]
            i0 = iv[0]
            i1 = iv[1] if K > 1 else i0
            lo = jnp.minimum(i0, i1)
            hi = jnp.maximum(i0, i1)
            base = g * P
            xsrc = xbuf.at[slot, pl.ds(t, 1)]

            zero_run(base, 0, lo, P - 1)
            pltpu.make_async_copy(
                xsrc, out_hbm.at[pl.ds(base + lo, 1)], wsem).start()
            zero_run(base, lo + 1, hi - lo - 1, P - 2)
            @pl.when(hi != lo)
            def _():
                pltpu.make_async_copy(
                    xsrc, out_hbm.at[pl.ds(base + hi, 1)], wsem).start()
            zero_run(base, hi + 1, (P - 1) - hi, P - 1)
            return carry

        lax.fori_loop(0, CHUNK, tok_body, 0)

        if ci + 1 < nchunks:
            if ci >= 1:
                drain_rows(CHUNK * P)
            load_chunk(ci + 1)
    drain_rows(min(2, nchunks) * CHUNK * P)


def kernel(x, partition_indices):
    B, S, D = x.shape
    T = B * S
    K = partition_indices.shape[-1]
    Tw = T // NW
    x2d = x.reshape(T, D)
    idxf = partition_indices.reshape(T * K).astype(jnp.int32)

    body = functools.partial(_sc_body, Tw, K, D)
    out = pl.kernel(
        body,
        out_type=jax.ShapeDtypeStruct((T * P, D), jnp.float32),
        mesh=plsc.VectorSubcoreMesh(core_axis_name="c", subcore_axis_name="s"),
        compiler_params=pltpu.CompilerParams(use_tc_tiling_on_sc=False),
        scratch_types=[
            pltpu.VMEM((2, CHUNK, D), jnp.float32),
            pltpu.VMEM((Tw * K + 16,), jnp.int32),
            pltpu.VMEM((P - 1, D), jnp.float32),
            pltpu.SemaphoreType.DMA,
            pltpu.SemaphoreType.DMA,
        ],
    )(x2d, idxf)
    return out.reshape(B, S, P, D)


# speedup vs baseline: 2.0713x; 2.0713x over previous
import jax
import jax.numpy as jnp
from jax.experimental import pallas as pl
from jax.experimental.pallas import tpu as pltpu

NUM_PARTITIONS = 8
TBLK = 256
NCH = 4
CH = TBLK // NCH


def _mask_bcast_kernel(idx_ref, x_ref, out_hbm, scratch, sems):
    i = pl.program_id(0)
    n = pl.num_programs(0)
    slot = jax.lax.rem(i, 2)
    K = idx_ref.shape[1]

    def wait_slot(s, step):
        for c in range(NCH):
            pltpu.make_async_copy(
                scratch.at[s, pl.ds(c * CH, CH)],
                out_hbm.at[pl.ds(step * TBLK + c * CH, CH)],
                sems.at[s, c],
            ).wait()

    @pl.when(i >= 2)
    def _():
        wait_slot(slot, i - 2)

    piota = jax.lax.broadcasted_iota(
        jnp.int32, (TBLK, NUM_PARTITIONS, 1), 1)
    m = idx_ref[:, 0:1, :] == piota
    for k in range(1, K):
        m = m | (idx_ref[:, k:k + 1, :] == piota)
    blk = jnp.where(m, x_ref[...], 0.0)

    @pl.when(slot == 0)
    def _():
        scratch[0] = blk

    @pl.when(slot == 1)
    def _():
        scratch[1] = blk

    for c in range(NCH):
        pltpu.make_async_copy(
            scratch.at[slot, pl.ds(c * CH, CH)],
            out_hbm.at[pl.ds(i * TBLK + c * CH, CH)],
            sems.at[slot, c],
        ).start()

    @pl.when(i == n - 1)
    def _():
        wait_slot(1 - slot, i - 1)
        wait_slot(slot, i)


def kernel(x, partition_indices):
    B, S, D = x.shape
    T = B * S
    K = partition_indices.shape[-1]
    xf = x.reshape(T, 1, D)
    idx = partition_indices.reshape(T, K, 1).astype(jnp.int32)

    out = pl.pallas_call(
        _mask_bcast_kernel,
        grid=(T // TBLK,),
        in_specs=[
            pl.BlockSpec((TBLK, K, 1), lambda i: (i, 0, 0)),
            pl.BlockSpec((TBLK, 1, D), lambda i: (i, 0, 0)),
        ],
        out_specs=pl.BlockSpec(memory_space=pl.ANY),
        out_shape=jax.ShapeDtypeStruct((T, NUM_PARTITIONS, D), x.dtype),
        scratch_shapes=[
            pltpu.VMEM((2, TBLK, NUM_PARTITIONS, D), x.dtype),
            pltpu.SemaphoreType.DMA((2, NCH)),
        ],
    )(idx, xf)
    return out.reshape(B, S, NUM_PARTITIONS, D)


# speedup vs baseline: 2.8065x; 1.3549x over previous
import functools

import jax
import jax.numpy as jnp
from jax import lax
from jax.experimental import pallas as pl
from jax.experimental.pallas import tpu as pltpu
from jax.experimental.pallas import tpu_sc as plsc

NUM_PARTITIONS = 8
P = NUM_PARTITIONS
NW = 32
CX = 32
NCHK = 4
ZR = 32


def _sc_body(Tw, K, D, x_hbm, rows_hbm, out_hbm,
             xbuf, idxv, zbuf, zs0, zs1, zs2, zs3, xs0, xs1, ss0, ss1):
    zsems = [zs0, zs1, zs2, zs3]
    xsems = [xs0, xs1]
    ssems = [ss0, ss1]
    wid = lax.axis_index("s") * 2 + lax.axis_index("c")
    tbase = wid * Tw
    NV = D // 16
    NZ = (CX * P) // ZR

    def zrow(r, carry):
        for v in range(NV):
            zbuf[r, pl.ds(v * 16, 16)] = jnp.zeros((16,), jnp.float32)
        return carry
    lax.fori_loop(0, ZR, zrow, 0)

    for k in range(K):
        pltpu.sync_copy(rows_hbm.at[k, wid], idxv.at[k])

    def zero_chunk(ci, do_start):
        base = (tbase + ci * CX) * P
        for j in range(NZ):
            cp = pltpu.make_async_copy(
                zbuf, out_hbm.at[pl.ds(base + j * ZR, ZR)], zsems[ci])
            if do_start:
                cp.start()
            else:
                cp.wait()

    def copy_x(ci, do_start):
        cp = pltpu.make_async_copy(
            x_hbm.at[pl.ds(tbase + ci * CX, CX)],
            xbuf.at[ci % 2], xsems[ci % 2])
        if do_start:
            cp.start()
        else:
            cp.wait()

    def scatter(ci, do_start):
        for k in range(K):
            cp = pltpu.make_async_copy(
                xbuf.at[ci % 2], out_hbm.at[idxv.at[k, ci]], ssems[ci % 2])
            if do_start:
                cp.start()
            else:
                cp.wait()

    zero_chunk(0, True)
    copy_x(0, True)
    for ci in range(NCHK):
        if ci + 1 < NCHK:
            zero_chunk(ci + 1, True)
            if ci >= 1:
                scatter(ci - 1, False)
            copy_x(ci + 1, True)
        zero_chunk(ci, False)
        copy_x(ci, False)
        scatter(ci, True)
    scatter(NCHK - 2, False)
    scatter(NCHK - 1, False)


def kernel(x, partition_indices):
    B, S, D = x.shape
    T = B * S
    K = partition_indices.shape[-1]
    Tw = T // NW
    x2d = x.reshape(T, D)
    idx = partition_indices.reshape(T, K).astype(jnp.int32)
    rows = jnp.arange(T, dtype=jnp.int32)[:, None] * P + idx
    rows = rows.T.reshape(K, NW, NCHK, CX)

    body = functools.partial(_sc_body, Tw, K, D)
    out = pl.kernel(
        body,
        out_type=jax.ShapeDtypeStruct((T * P, D), jnp.float32),
        mesh=plsc.VectorSubcoreMesh(core_axis_name="c", subcore_axis_name="s"),
        scratch_types=[
            pltpu.VMEM((2, CX, D), jnp.float32),
            pltpu.VMEM((K, NCHK, CX), jnp.int32),
            pltpu.VMEM((ZR, D), jnp.float32),
            pltpu.SemaphoreType.DMA,
            pltpu.SemaphoreType.DMA,
            pltpu.SemaphoreType.DMA,
            pltpu.SemaphoreType.DMA,
            pltpu.SemaphoreType.DMA,
            pltpu.SemaphoreType.DMA,
            pltpu.SemaphoreType.DMA,
            pltpu.SemaphoreType.DMA,
        ],
    )(x2d, rows)
    return out.reshape(B, S, P, D)
